# baseline (speedup 1.0000x reference)
import functools

import jax
import jax.numpy as jnp
from jax import lax
from jax.experimental import pallas as pl
from jax.experimental.pallas import tpu as pltpu
from jax.experimental.pallas import tpu_sc as plsc

_GATHER_DN = lax.GatherDimensionNumbers(
    offset_dims=(), collapsed_slice_dims=(0,), start_index_map=(0,))


def _lane_bcast(vec, lane):
    idx = jnp.full((16, 1), lane, jnp.int32)
    return lax.gather(vec, idx, _GATHER_DN, slice_sizes=(1,),
                      mode=lax.GatherScatterMode.PROMISE_IN_BOUNDS)


N_NODES = 2708
N_EDGES = 13264
D_IN = 1433
D_OUT = 16

NT = 16
NC = 2
CH = 128
NCH = 7
E_TILE = CH * NCH
E_PAD = E_TILE * NT
N_PAD = 2816
ROWS_INIT = N_PAD // NT
ROWS_OUT = N_PAD // (NT * NC)


def _mm_body(xt_ref, w_ref, o_ref):
    o_ref[...] = lax.dot_general(
        xt_ref[...], w_ref[...],
        dimension_numbers=(((0,), (1,)), ((), ())),
        preferred_element_type=jnp.float32,
    )


def _matmul(xt, w):
    blk = 512
    grid = (N_NODES + blk - 1) // blk
    return pl.pallas_call(
        _mm_body,
        grid=(grid,),
        in_specs=[
            pl.BlockSpec((D_IN, blk), lambda i: (0, i)),
            pl.BlockSpec((D_OUT, D_IN), lambda i: (0, 0)),
        ],
        out_specs=pl.BlockSpec((blk, D_OUT), lambda i: (i, 0)),
        out_shape=jax.ShapeDtypeStruct((N_NODES, D_OUT), jnp.float32),
    )(xt, w)


def _sc_aggregate_kernel(mm_hbm, src_hbm, dst_hbm, w_hbm, bias_hbm, out_hbm,
                         src_v, dst_v, w_v, rows_v, bias_v, biasblk_v,
                         acc_shared, sem, sem_g, sem_s):
    c = lax.axis_index("c")
    s = lax.axis_index("s")

    pltpu.sync_copy(src_hbm.at[s], src_v)
    gathers = [
        pltpu.async_copy(mm_hbm.at[src_v.at[ch]],
                         rows_v.at[pl.ds(ch * CH, CH)], sem_g)
        for ch in range(NCH)
    ]
    dcp = pltpu.async_copy(dst_hbm.at[s], dst_v, sem)
    wcp = pltpu.async_copy(w_hbm.at[s], w_v, sem)

    pltpu.sync_copy(bias_hbm, bias_v)
    bvec = bias_v[...]

    def _initrow(r, carry):
        biasblk_v[r, :] = bvec
        return carry

    lax.fori_loop(0, ROWS_INIT, _initrow, 0)
    pltpu.sync_copy(biasblk_v, acc_shared.at[pl.ds(s * ROWS_INIT, ROWS_INIT)])
    dcp.wait()
    wcp.wait()
    plsc.subcore_barrier()

    scatters = []
    for ch in range(NCH):
        gathers[ch].wait()

        def _group(g, carry, ch=ch):
            wvec = w_v[ch, pl.ds(g * 16, 16)]
            for e in range(16):
                r = ch * CH + g * 16 + e
                rows_v[r, :] = rows_v[r, :] * _lane_bcast(wvec, e)
            return carry

        lax.fori_loop(0, CH // 16, _group, 0)
        scatters.append(
            pltpu.async_copy(rows_v.at[pl.ds(ch * CH, CH)],
                             acc_shared.at[dst_v.at[ch]], sem_s, add=True))

    for cp in scatters:
        cp.wait()
    plsc.subcore_barrier()

    row0 = (c * NT + s) * ROWS_OUT
    pltpu.sync_copy(acc_shared.at[pl.ds(row0, ROWS_OUT)],
                    out_hbm.at[pl.ds(row0, ROWS_OUT)])


def _sc_aggregate(mm, src_r, dst_r, w_r, bias):
    mesh = plsc.VectorSubcoreMesh(core_axis_name="c", subcore_axis_name="s")
    kern = functools.partial(
        pl.kernel,
        mesh=mesh,
        compiler_params=pltpu.CompilerParams(use_tc_tiling_on_sc=False),
        out_type=jax.ShapeDtypeStruct((N_PAD, D_OUT), jnp.float32),
        scratch_types=[
            pltpu.VMEM((NCH, CH), jnp.int32),
            pltpu.VMEM((NCH, CH), jnp.int32),
            pltpu.VMEM((NCH, CH), jnp.float32),
            pltpu.VMEM((NCH * CH, D_OUT), jnp.float32),
            pltpu.VMEM((D_OUT,), jnp.float32),
            pltpu.VMEM((ROWS_INIT, D_OUT), jnp.float32),
            pltpu.VMEM_SHARED((N_PAD, D_OUT), jnp.float32),
            pltpu.SemaphoreType.DMA,
            pltpu.SemaphoreType.DMA,
            pltpu.SemaphoreType.DMA,
        ],
    )(_sc_aggregate_kernel)
    return kern(mm, src_r, dst_r, w_r, bias)


def kernel(arg0_1, arg1_1, arg2_1, arg3_1, arg4_1):
    mm = _matmul(arg4_1.T, arg0_1)

    src = arg2_1[0].astype(jnp.int32)
    dst = arg2_1[1].astype(jnp.int32)
    w = arg3_1.astype(jnp.float32)
    pad = E_PAD - N_EDGES
    src_r = jnp.pad(src, (0, pad)).reshape(NT, NCH, CH)
    dst_r = jnp.pad(dst, (0, pad)).reshape(NT, NCH, CH)
    w_r = jnp.pad(w, (0, pad)).reshape(NT, NCH, CH)

    out = _sc_aggregate(mm, src_r, dst_r, w_r, arg1_1)
    return out[:N_NODES]


# speedup vs baseline: 1.2479x; 1.2479x over previous
import functools

import jax
import jax.numpy as jnp
from jax import lax
from jax.experimental import pallas as pl
from jax.experimental.pallas import tpu as pltpu
from jax.experimental.pallas import tpu_sc as plsc

_GATHER_DN = lax.GatherDimensionNumbers(
    offset_dims=(), collapsed_slice_dims=(0,), start_index_map=(0,))


def _lane_bcast(vec, lane):
    idx = jnp.full((16, 1), lane, jnp.int32)
    return lax.gather(vec, idx, _GATHER_DN, slice_sizes=(1,),
                      mode=lax.GatherScatterMode.PROMISE_IN_BOUNDS)


N_NODES = 2708
N_EDGES = 13264
D_IN = 1433
D_OUT = 16

NT = 16
NC = 2
CH = 128
NCH = 7
E_TILE = CH * NCH
E_PAD = E_TILE * NT
N_PAD = 2816
ROWS_INIT = N_PAD // NT
ROWS_OUT = N_PAD // (NT * NC)


def _mm_body(xt_ref, w_ref, o_ref):
    o_ref[...] = lax.dot_general(
        xt_ref[...], w_ref[...],
        dimension_numbers=(((0,), (1,)), ((), ())),
        preferred_element_type=jnp.float32,
    )


def _matmul(xt, w):
    blk = 1408
    grid = N_PAD // blk
    return pl.pallas_call(
        _mm_body,
        grid=(grid,),
        in_specs=[
            pl.BlockSpec((D_IN, blk), lambda i: (0, i)),
            pl.BlockSpec((D_OUT, D_IN), lambda i: (0, 0)),
        ],
        out_specs=pl.BlockSpec((blk, D_OUT), lambda i: (i, 0)),
        out_shape=jax.ShapeDtypeStruct((N_PAD, D_OUT), jnp.float32),
    )(xt, w)


def _sc_aggregate_kernel(mm_hbm, src_hbm, dst_hbm, w_hbm, bias_hbm, out_hbm,
                         src_v, dst_v, w_v, rows_v, bias_v, biasblk_v,
                         acc_shared, mm_shared, sem, sem_g, sem_s):
    c = lax.axis_index("c")
    s = lax.axis_index("s")

    scp = pltpu.async_copy(src_hbm.at[s], src_v, sem)
    mcp = pltpu.async_copy(mm_hbm.at[pl.ds(s * ROWS_INIT, ROWS_INIT)],
                           mm_shared.at[pl.ds(s * ROWS_INIT, ROWS_INIT)],
                           sem_g)
    dcp = pltpu.async_copy(dst_hbm.at[s], dst_v, sem)
    wcp = pltpu.async_copy(w_hbm.at[s], w_v, sem)

    pltpu.sync_copy(bias_hbm, bias_v)
    bvec = bias_v[...]

    def _initrow(r, carry):
        biasblk_v[r, :] = bvec
        return carry

    lax.fori_loop(0, ROWS_INIT, _initrow, 0)
    pltpu.sync_copy(biasblk_v, acc_shared.at[pl.ds(s * ROWS_INIT, ROWS_INIT)])
    scp.wait()
    dcp.wait()
    wcp.wait()
    mcp.wait()
    plsc.subcore_barrier()

    gathers = [
        pltpu.async_copy(mm_shared.at[src_v.at[ch]],
                         rows_v.at[pl.ds(ch * CH, CH)], sem_g)
        for ch in range(NCH)
    ]
    scatters = []
    for ch in range(NCH):
        gathers[ch].wait()

        def _group(g, carry, ch=ch):
            wvec = w_v[ch, pl.ds(g * 16, 16)]
            for e in range(16):
                r = ch * CH + g * 16 + e
                rows_v[r, :] = rows_v[r, :] * _lane_bcast(wvec, e)
            return carry

        lax.fori_loop(0, CH // 16, _group, 0)
        scatters.append(
            pltpu.async_copy(rows_v.at[pl.ds(ch * CH, CH)],
                             acc_shared.at[dst_v.at[ch]], sem_s, add=True))

    for cp in scatters:
        cp.wait()
    plsc.subcore_barrier()

    row0 = (c * NT + s) * ROWS_OUT
    pltpu.sync_copy(acc_shared.at[pl.ds(row0, ROWS_OUT)],
                    out_hbm.at[pl.ds(row0, ROWS_OUT)])


def _sc_aggregate(mm, src_r, dst_r, w_r, bias):
    mesh = plsc.VectorSubcoreMesh(core_axis_name="c", subcore_axis_name="s")
    kern = functools.partial(
        pl.kernel,
        mesh=mesh,
        compiler_params=pltpu.CompilerParams(use_tc_tiling_on_sc=False),
        out_type=jax.ShapeDtypeStruct((N_PAD, D_OUT), jnp.float32),
        scratch_types=[
            pltpu.VMEM((NCH, CH), jnp.int32),
            pltpu.VMEM((NCH, CH), jnp.int32),
            pltpu.VMEM((NCH, CH), jnp.float32),
            pltpu.VMEM((NCH * CH, D_OUT), jnp.float32),
            pltpu.VMEM((D_OUT,), jnp.float32),
            pltpu.VMEM((ROWS_INIT, D_OUT), jnp.float32),
            pltpu.VMEM_SHARED((N_PAD, D_OUT), jnp.float32),
            pltpu.VMEM_SHARED((N_PAD, D_OUT), jnp.float32),
            pltpu.SemaphoreType.DMA,
            pltpu.SemaphoreType.DMA,
            pltpu.SemaphoreType.DMA,
        ],
    )(_sc_aggregate_kernel)
    return kern(mm, src_r, dst_r, w_r, bias)


def kernel(arg0_1, arg1_1, arg2_1, arg3_1, arg4_1):
    mm = _matmul(arg4_1.T, arg0_1)

    src = arg2_1[0].astype(jnp.int32)
    dst = arg2_1[1].astype(jnp.int32)
    w = arg3_1.astype(jnp.float32)
    pad = E_PAD - N_EDGES
    src_r = jnp.pad(src, (0, pad)).reshape(NT, NCH, CH)
    dst_r = jnp.pad(dst, (0, pad)).reshape(NT, NCH, CH)
    w_r = jnp.pad(w, (0, pad)).reshape(NT, NCH, CH)

    out = _sc_aggregate(mm, src_r, dst_r, w_r, arg1_1)
    return out[:N_NODES]


# speedup vs baseline: 1.3052x; 1.0460x over previous
import functools

import jax
import jax.numpy as jnp
from jax import lax
from jax.experimental import pallas as pl
from jax.experimental.pallas import tpu as pltpu
from jax.experimental.pallas import tpu_sc as plsc

_GATHER_DN = lax.GatherDimensionNumbers(
    offset_dims=(), collapsed_slice_dims=(0,), start_index_map=(0,))


def _lane_bcast(vec, lane):
    idx = jnp.full((16, 1), lane, jnp.int32)
    return lax.gather(vec, idx, _GATHER_DN, slice_sizes=(1,),
                      mode=lax.GatherScatterMode.PROMISE_IN_BOUNDS)


N_NODES = 2708
N_EDGES = 13264
D_IN = 1433
D_OUT = 16

NT = 16
NC = 2
CH = 64
NCH = 13
E_TILE = CH * NCH
N_PAD = 2816
DUMMY = N_NODES
ROWS_INIT = N_PAD // NT
ROWS_OUT = N_PAD // (NT * NC)


def _mm_body(xt_ref, w_ref, o_ref):
    o_ref[...] = lax.dot_general(
        xt_ref[...], w_ref[...],
        dimension_numbers=(((0,), (1,)), ((), ())),
        preferred_element_type=jnp.float32,
    )


def _matmul(xt, w):
    blk = 1408
    grid = N_PAD // blk
    return pl.pallas_call(
        _mm_body,
        grid=(grid,),
        in_specs=[
            pl.BlockSpec((D_IN, blk), lambda i: (0, i)),
            pl.BlockSpec((D_OUT, D_IN), lambda i: (0, 0)),
        ],
        out_specs=pl.BlockSpec((blk, D_OUT), lambda i: (i, 0)),
        out_shape=jax.ShapeDtypeStruct((N_PAD, D_OUT), jnp.float32),
    )(xt, w)


def _sc_aggregate_kernel(mm_hbm, edges_hbm, w_hbm, bias_hbm, out_hbm,
                         src_v, dst_v, w_v, dst2_v, rows_v, bias_v, biasblk_v,
                         acc_shared, mm_shared, sem, sem_g, sem_s):
    c = lax.axis_index("c")
    s = lax.axis_index("s")
    a = pl.multiple_of(s * E_TILE, 8)

    scp = pltpu.async_copy(edges_hbm.at[0, pl.ds(a, E_TILE)], src_v, sem)
    mcp = pltpu.async_copy(mm_hbm.at[pl.ds(s * ROWS_INIT, ROWS_INIT)],
                           mm_shared.at[pl.ds(s * ROWS_INIT, ROWS_INIT)],
                           sem_g)
    dcp = pltpu.async_copy(edges_hbm.at[1, pl.ds(a, E_TILE)], dst_v, sem)
    wcp = pltpu.async_copy(w_hbm.at[pl.ds(a, E_TILE)], w_v, sem)

    pltpu.sync_copy(bias_hbm, bias_v)
    bvec = bias_v[...]

    def _initrow(r, carry):
        biasblk_v[r, :] = bvec
        return carry

    lax.fori_loop(0, ROWS_INIT, _initrow, 0)
    pltpu.sync_copy(biasblk_v, acc_shared.at[pl.ds(s * ROWS_INIT, ROWS_INIT)])
    scp.wait()
    dcp.wait()
    wcp.wait()

    lanes = lax.iota(jnp.int32, 16)
    zeros16 = jnp.zeros((16,), jnp.int32)
    dummy16 = jnp.full((16,), DUMMY, jnp.int32)
    nedges16 = jnp.full((16,), N_EDGES, jnp.int32)
    abase = jnp.full((16,), a, jnp.int32)

    def _sanitize(g, carry):
        off = g * 16
        eidx = abase + off + lanes
        ok = eidx < nedges16
        svec = jnp.where(ok, src_v[pl.ds(off, 16)], zeros16)
        dvec = jnp.where(ok, dst_v[pl.ds(off, 16)], dummy16)
        src_v[pl.ds(off, 16)] = svec
        dst2_v[g // (CH // 16), pl.ds((g % (CH // 16)) * 16, 16)] = dvec
        return carry

    lax.fori_loop(0, E_TILE // 16, _sanitize, 0)

    mcp.wait()
    plsc.subcore_barrier()

    gathers = [
        pltpu.async_copy(mm_shared.at[src_v.at[pl.ds(ch * CH, CH)]],
                         rows_v.at[pl.ds(ch * CH, CH)], sem_g)
        for ch in range(NCH)
    ]
    scatters = []
    for ch in range(NCH):
        gathers[ch].wait()

        def _group(g, carry, ch=ch):
            wvec = w_v[pl.ds(ch * CH + g * 16, 16)]
            for e in range(16):
                r = ch * CH + g * 16 + e
                rows_v[r, :] = rows_v[r, :] * _lane_bcast(wvec, e)
            return carry

        lax.fori_loop(0, CH // 16, _group, 0)
        scatters.append(
            pltpu.async_copy(rows_v.at[pl.ds(ch * CH, CH)],
                             acc_shared.at[dst2_v.at[ch]], sem_s, add=True))

    for cp in scatters:
        cp.wait()
    plsc.subcore_barrier()

    row0 = (c * NT + s) * ROWS_OUT
    pltpu.sync_copy(acc_shared.at[pl.ds(row0, ROWS_OUT)],
                    out_hbm.at[pl.ds(row0, ROWS_OUT)])


def _sc_aggregate(mm, edges, w, bias):
    mesh = plsc.VectorSubcoreMesh(core_axis_name="c", subcore_axis_name="s")
    kern = functools.partial(
        pl.kernel,
        mesh=mesh,
        compiler_params=pltpu.CompilerParams(use_tc_tiling_on_sc=False),
        out_type=jax.ShapeDtypeStruct((N_PAD, D_OUT), jnp.float32),
        scratch_types=[
            pltpu.VMEM((E_TILE,), jnp.int32),
            pltpu.VMEM((E_TILE,), jnp.int32),
            pltpu.VMEM((E_TILE,), jnp.float32),
            pltpu.VMEM((NCH, CH), jnp.int32),
            pltpu.VMEM((E_TILE, D_OUT), jnp.float32),
            pltpu.VMEM((D_OUT,), jnp.float32),
            pltpu.VMEM((ROWS_INIT, D_OUT), jnp.float32),
            pltpu.VMEM_SHARED((N_PAD, D_OUT), jnp.float32),
            pltpu.VMEM_SHARED((N_PAD, D_OUT), jnp.float32),
            pltpu.SemaphoreType.DMA,
            pltpu.SemaphoreType.DMA,
            pltpu.SemaphoreType.DMA,
        ],
    )(_sc_aggregate_kernel)
    return kern(mm, edges, w, bias)


def kernel(arg0_1, arg1_1, arg2_1, arg3_1, arg4_1):
    mm = _matmul(arg4_1.T, arg0_1)
    out = _sc_aggregate(mm, arg2_1.astype(jnp.int32), arg3_1, arg1_1)
    return out[:N_NODES]
